# Initial kernel scaffold; baseline (speedup 1.0000x reference)
import jax
import jax.numpy as jnp
from jax.experimental import pallas as pl


def kernel(features, edge_index, edge_values, weight):
    raise NotImplementedError("write your pallas kernel here")


# speedup vs baseline: 6.4154x; 6.4154x over previous
import jax
import jax.numpy as jnp
from jax import lax
from jax.experimental import pallas as pl
from jax.experimental.pallas import tpu as pltpu
from jax.experimental.pallas import tpu_sc as plsc

_NC = 2
_NS = 16

_N = 10000
_E = 320000
_D = 128

_EPW = _E // (_NC * _NS)
_CH = 80
_NCH = _EPW // _CH
_BCH = 25
_NBLK = _NCH // _BCH
_NPAD = 10240
_RPT = _NPAD // _NS
_ZR = 128


def _mm_body(f_ref, w_ref, o_ref):
    o_ref[...] = jnp.dot(f_ref[...], w_ref[...], preferred_element_type=jnp.float32)


def _matmul(features, weight):
    n, d_in = features.shape
    d_out = weight.shape[1]
    blk = 1000
    return pl.pallas_call(
        _mm_body,
        grid=(n // blk,),
        in_specs=[
            pl.BlockSpec((blk, d_in), lambda i: (i, 0)),
            pl.BlockSpec((d_in, d_out), lambda i: (0, 0)),
        ],
        out_specs=pl.BlockSpec((blk, d_out), lambda i: (i, 0)),
        out_shape=jax.ShapeDtypeStruct((n, d_out), jnp.float32),
    )(features, weight)


def _comb_body(p0_ref, p1_ref, o_ref):
    o_ref[...] = jnp.maximum(p0_ref[...] + p1_ref[...], 0.0)


def _combine(p0, p1, n):
    blk = 1000
    return pl.pallas_call(
        _comb_body,
        grid=(n // blk,),
        in_specs=[
            pl.BlockSpec((blk, _D), lambda i: (i, 0)),
            pl.BlockSpec((blk, _D), lambda i: (i, 0)),
        ],
        out_specs=pl.BlockSpec((blk, _D), lambda i: (i, 0)),
        out_shape=jax.ShapeDtypeStruct((n, _D), jnp.float32),
    )(p0, p1)


def _spmm_body(sup, colh, rowh, evh, out0, out1,
               colb, rowb, evb, rowv, msgs, zbuf, acc, gsem):
    c = lax.axis_index("c")
    s = lax.axis_index("s")

    def _zrow(r, carry):
        for d in range(_D // 16):
            zbuf[r, pl.ds(d * 16, 16)] = jnp.zeros((16,), jnp.float32)
        return carry

    lax.fori_loop(0, _ZR, _zrow, 0)
    for j in range(_RPT // _ZR):
        pltpu.sync_copy(zbuf, acc.at[pl.ds(s * _RPT + j * _ZR, _ZR)])
    plsc.subcore_barrier()

    def _block(b, carry):
        pltpu.sync_copy(colh.at[c, s, b, pl.ds(0, _BCH), pl.ds(0, _CH)], colb)
        pltpu.sync_copy(rowh.at[c, s, b, pl.ds(0, _BCH), pl.ds(0, _CH)], rowb)
        pltpu.sync_copy(evh.at[c, s, b, pl.ds(0, _BCH), pl.ds(0, _CH)], evb)

        def _chunk(q, carry2):
            pltpu.async_copy(sup.at[colb.at[q]], msgs, gsem).wait()
            for k in range(_CH // 16):
                rowv[pl.ds(k * 16, 16)] = rowb[q, pl.ds(k * 16, 16)]
                evvec = evb[q, pl.ds(k * 16, 16)]
                for lane in range(16):
                    e = k * 16 + lane
                    sv = evvec[lane]
                    for d in range(_D // 16):
                        msgs[e, pl.ds(d * 16, 16)] = (
                            msgs[e, pl.ds(d * 16, 16)] * sv
                        )
            pltpu.sync_copy(msgs, acc.at[rowv], add=True)
            return carry2

        lax.fori_loop(0, _BCH, _chunk, 0)
        return carry

    lax.fori_loop(0, _NBLK, _block, 0)
    plsc.subcore_barrier()

    for j in range(_RPT // _ZR):
        pltpu.sync_copy(acc.at[pl.ds(s * _RPT + j * _ZR, _ZR)], zbuf)

        @pl.when(c == 0)
        def _():
            pltpu.sync_copy(zbuf, out0.at[pl.ds(s * _RPT + j * _ZR, _ZR)])

        @pl.when(c == 1)
        def _():
            pltpu.sync_copy(zbuf, out1.at[pl.ds(s * _RPT + j * _ZR, _ZR)])


_SCRATCH = [
    pltpu.VMEM((_BCH, _CH), jnp.int32),
    pltpu.VMEM((_BCH, _CH), jnp.int32),
    pltpu.VMEM((_BCH, _CH), jnp.float32),
    pltpu.VMEM((_CH,), jnp.int32),
    pltpu.VMEM((_CH, _D), jnp.float32),
    pltpu.VMEM((_ZR, _D), jnp.float32),
    pltpu.VMEM_SHARED((_NPAD, _D), jnp.float32),
    pltpu.SemaphoreType.DMA,
]

_spmm = pl.kernel(
    _spmm_body,
    out_type=(
        jax.ShapeDtypeStruct((_NPAD, _D), jnp.float32),
        jax.ShapeDtypeStruct((_NPAD, _D), jnp.float32),
    ),
    mesh=plsc.VectorSubcoreMesh(core_axis_name="c", subcore_axis_name="s"),
    scratch_types=_SCRATCH,
)


def kernel(features, edge_index, edge_values, weight):
    sup = _matmul(features, weight)
    col = edge_index[1].reshape(_NC, _NS, _NBLK, _BCH, _CH)
    row = edge_index[0].reshape(_NC, _NS, _NBLK, _BCH, _CH)
    ev = edge_values.reshape(_NC, _NS, _NBLK, _BCH, _CH)
    p0, p1 = _spmm(sup, col, row, ev)
    return _combine(p0, p1, features.shape[0])


# speedup vs baseline: 9.7587x; 1.5211x over previous
import jax
import jax.numpy as jnp
from jax import lax
from jax.experimental import pallas as pl
from jax.experimental.pallas import tpu as pltpu
from jax.experimental.pallas import tpu_sc as plsc

_NC = 2
_NS = 16

_N = 10000
_E = 320000
_D = 128

_EPW = _E // (_NC * _NS)
_CH = 80
_NCH = _EPW // _CH
_BCH = 25
_NBLK = _NCH // _BCH
_NPAD = 10112
_RPT = _NPAD // _NS
_ZR = 128
_SEGS = [(j * _ZR, _ZR) for j in range(_RPT // _ZR)]
if _RPT % _ZR:
    _SEGS.append(((_RPT // _ZR) * _ZR, _RPT % _ZR))


def _mm_body(f_ref, w_ref, o_ref):
    o_ref[...] = jnp.dot(f_ref[...], w_ref[...], preferred_element_type=jnp.float32)


def _matmul(features, weight):
    n, d_in = features.shape
    d_out = weight.shape[1]
    blk = 1000
    return pl.pallas_call(
        _mm_body,
        grid=(n // blk,),
        in_specs=[
            pl.BlockSpec((blk, d_in), lambda i: (i, 0)),
            pl.BlockSpec((d_in, d_out), lambda i: (0, 0)),
        ],
        out_specs=pl.BlockSpec((blk, d_out), lambda i: (i, 0)),
        out_shape=jax.ShapeDtypeStruct((n, d_out), jnp.float32),
    )(features, weight)


def _comb_body(p0_ref, p1_ref, o_ref):
    o_ref[...] = jnp.maximum(p0_ref[...] + p1_ref[...], 0.0)


def _combine(p0, p1, n):
    blk = 1000
    return pl.pallas_call(
        _comb_body,
        grid=(n // blk,),
        in_specs=[
            pl.BlockSpec((blk, _D), lambda i: (i, 0)),
            pl.BlockSpec((blk, _D), lambda i: (i, 0)),
        ],
        out_specs=pl.BlockSpec((blk, _D), lambda i: (i, 0)),
        out_shape=jax.ShapeDtypeStruct((n, _D), jnp.float32),
    )(p0, p1)


def _spmm_body(sup, colh, rowh, evh, out0, out1,
               colb, rowb, evb, rowv, msgs, zbuf, acc, gsem0, gsem1):
    c = lax.axis_index("c")
    s = lax.axis_index("s")

    def _zrow(r, carry):
        for d in range(_D // 16):
            zbuf[r, pl.ds(d * 16, 16)] = jnp.zeros((16,), jnp.float32)
        return carry

    lax.fori_loop(0, _ZR, _zrow, 0)
    for off, ln in _SEGS:
        pltpu.sync_copy(zbuf.at[pl.ds(0, ln)], acc.at[pl.ds(s * _RPT + off, ln)])
    plsc.subcore_barrier()

    def _block(b, carry):
        pltpu.sync_copy(colh.at[c, s, b, pl.ds(0, _BCH), pl.ds(0, _CH)], colb)
        pltpu.sync_copy(rowh.at[c, s, b, pl.ds(0, _BCH), pl.ds(0, _CH)], rowb)
        pltpu.sync_copy(evh.at[c, s, b, pl.ds(0, _BCH), pl.ds(0, _CH)], evb)

        pltpu.async_copy(sup.at[colb.at[0]], msgs.at[0], gsem0)

        def _chunk(q, carry2):
            p = lax.rem(q, 2)

            @pl.when(jnp.logical_and(q + 1 < _BCH, p == 0))
            def _():
                pltpu.async_copy(sup.at[colb.at[q + 1]], msgs.at[1], gsem1)

            @pl.when(jnp.logical_and(q + 1 < _BCH, p == 1))
            def _():
                pltpu.async_copy(sup.at[colb.at[q + 1]], msgs.at[0], gsem0)

            @pl.when(p == 0)
            def _():
                pltpu.make_async_copy(
                    sup.at[pl.ds(0, _CH)], msgs.at[0], gsem0).wait()

            @pl.when(p == 1)
            def _():
                pltpu.make_async_copy(
                    sup.at[pl.ds(0, _CH)], msgs.at[1], gsem1).wait()

            for k in range(_CH // 16):
                rowv[pl.ds(k * 16, 16)] = rowb[q, pl.ds(k * 16, 16)]
                evvec = evb[q, pl.ds(k * 16, 16)]
                for lane in range(16):
                    e = k * 16 + lane
                    sv = evvec[lane]
                    for d in range(_D // 16):
                        msgs[p, e, pl.ds(d * 16, 16)] = (
                            msgs[p, e, pl.ds(d * 16, 16)] * sv
                        )
            pltpu.sync_copy(msgs.at[p], acc.at[rowv], add=True)
            return carry2

        lax.fori_loop(0, _BCH, _chunk, 0)
        return carry

    lax.fori_loop(0, _NBLK, _block, 0)
    plsc.subcore_barrier()

    for off, ln in _SEGS:
        pltpu.sync_copy(acc.at[pl.ds(s * _RPT + off, ln)], zbuf.at[pl.ds(0, ln)])

        @pl.when(c == 0)
        def _():
            pltpu.sync_copy(zbuf.at[pl.ds(0, ln)],
                            out0.at[pl.ds(s * _RPT + off, ln)])

        @pl.when(c == 1)
        def _():
            pltpu.sync_copy(zbuf.at[pl.ds(0, ln)],
                            out1.at[pl.ds(s * _RPT + off, ln)])


_SCRATCH = [
    pltpu.VMEM((_BCH, _CH), jnp.int32),
    pltpu.VMEM((_BCH, _CH), jnp.int32),
    pltpu.VMEM((_BCH, _CH), jnp.float32),
    pltpu.VMEM((_CH,), jnp.int32),
    pltpu.VMEM((2, _CH, _D), jnp.float32),
    pltpu.VMEM((_ZR, _D), jnp.float32),
    pltpu.VMEM_SHARED((_NPAD, _D), jnp.float32),
    pltpu.SemaphoreType.DMA,
    pltpu.SemaphoreType.DMA,
]

_spmm = pl.kernel(
    _spmm_body,
    out_type=(
        jax.ShapeDtypeStruct((_NPAD, _D), jnp.float32),
        jax.ShapeDtypeStruct((_NPAD, _D), jnp.float32),
    ),
    mesh=plsc.VectorSubcoreMesh(core_axis_name="c", subcore_axis_name="s"),
    scratch_types=_SCRATCH,
)


def kernel(features, edge_index, edge_values, weight):
    sup = _matmul(features, weight)
    col = edge_index[1].reshape(_NC, _NS, _NBLK, _BCH, _CH)
    row = edge_index[0].reshape(_NC, _NS, _NBLK, _BCH, _CH)
    ev = edge_values.reshape(_NC, _NS, _NBLK, _BCH, _CH)
    p0, p1 = _spmm(sup, col, row, ev)
    return _combine(p0, p1, features.shape[0])
